# baseline (speedup 1.0000x reference)
import functools

import jax
import jax.numpy as jnp
from jax import lax
from jax.experimental import pallas as pl
from jax.experimental.pallas import tpu as pltpu
from jax.experimental.pallas import tpu_sc as plsc

NUM_CLASSES = 12
NFEAT = 16
NPROP = NFEAT - 1
NOUT = NUM_CLASSES + NPROP
LANES = 16
NUM_WORKERS = 32


def _sc_body(x_hbm, out_hbm, in_v, out_v, rows_per_worker):
    cid = lax.axis_index("c")
    sid = lax.axis_index("s")
    wid = sid * 2 + cid

    in_words = rows_per_worker * NFEAT
    out_words = rows_per_worker * NOUT

    pltpu.sync_copy(x_hbm.at[pl.ds(wid * in_words, in_words)], in_v)

    lane = lax.iota(jnp.int32, LANES)
    lane_f = lane.astype(jnp.float32)
    one = jnp.full((LANES,), 1.0, jnp.float32)
    zero = jnp.zeros((LANES,), jnp.float32)
    maskB = lane < (NOUT - LANES)
    in_lanes = lane < NFEAT

    def body(r, _):
        r16 = r * NFEAT
        r27 = r * NOUT
        idf = plsc.load_gather(in_v, [jnp.full((LANES,), r16, jnp.int32)])
        idi = idf.astype(jnp.int32)
        onehot = jnp.where(lane == idi, one, zero)
        gA = plsc.load_gather(
            in_v, [jnp.maximum(r16 - (NUM_CLASSES - 1) + lane, 0)]
        )
        a = jnp.where(lane < NUM_CLASSES, onehot, gA)
        gB = plsc.load_gather(
            in_v, [jnp.minimum(r16 + (LANES - NPROP + 4) + lane, in_words - 1)]
        )
        plsc.store_scatter(out_v, [r27 + lane], a)
        plsc.store_scatter(out_v, [r27 + LANES + lane], gB, mask=maskB)
        return ()

    plsc.parallel_loop(0, rows_per_worker, 1, unroll=8, carry=())(body)

    pltpu.sync_copy(out_v, out_hbm.at[pl.ds(wid * out_words, out_words)])


def kernel(X):
    B, N, F = X.shape
    assert F == NFEAT
    rows = B * N
    rpw = rows // NUM_WORKERS
    assert rpw * NUM_WORKERS == rows

    x_flat = X.reshape(rows * NFEAT)
    mesh = plsc.VectorSubcoreMesh(core_axis_name="c", subcore_axis_name="s")
    out_flat = pl.kernel(
        functools.partial(_sc_body, rows_per_worker=rpw),
        out_type=jax.ShapeDtypeStruct((rows * NOUT,), jnp.float32),
        mesh=mesh,
        compiler_params=pltpu.CompilerParams(needs_layout_passes=False),
        scratch_types=[
            pltpu.VMEM((rpw * NFEAT,), jnp.float32),
            pltpu.VMEM((rpw * NOUT,), jnp.float32),
        ],
    )(x_flat)
    return out_flat.reshape(B, N, NOUT)


# speedup vs baseline: 1.2287x; 1.2287x over previous
import functools

import jax
import jax.numpy as jnp
from jax import lax
from jax.experimental import pallas as pl
from jax.experimental.pallas import tpu as pltpu
from jax.experimental.pallas import tpu_sc as plsc

NUM_CLASSES = 12
NFEAT = 16
NPROP = NFEAT - 1
NOUT = NUM_CLASSES + NPROP
STRIDE = 32
LANES = 16
PAD = 16
NUM_WORKERS = 32


def _sc_body(x_hbm, out_hbm, in_v, st_v, rows_per_worker):
    cid = lax.axis_index("c")
    sid = lax.axis_index("s")
    wid = sid * 2 + cid
    row0 = wid * rows_per_worker
    nin = rows_per_worker * NFEAT

    pltpu.sync_copy(x_hbm.at[pl.ds(row0 * NFEAT, nin)], in_v.at[pl.ds(PAD, nin)])

    lane = lax.iota(jnp.int32, LANES)
    lane_f = lane.astype(jnp.float32)
    one = jnp.full((LANES,), 1.0, jnp.float32)
    zero = jnp.zeros((LANES,), jnp.float32)
    is_oh = lane < NUM_CLASSES
    eleven = jnp.full((LANES,), 11, jnp.int32)
    mask_b = lane < (NOUT - LANES)
    col_b = lane + LANES
    idx0 = PAD - (NUM_CLASSES - 1) + lane

    def body(r, idx):
        ga = plsc.load_gather(in_v, [idx])
        gb = plsc.load_gather(in_v, [idx + NFEAT])
        idb = ga.at[eleven].get(mode="promise_in_bounds")
        a = jnp.where(is_oh, jnp.where(lane_f == idb, one, zero), ga)
        st_v[r, pl.ds(0, LANES)] = a
        plsc.store_scatter(
            st_v, [jnp.full((LANES,), r, jnp.int32), col_b], gb, mask=mask_b
        )
        return idx + NFEAT

    plsc.parallel_loop(0, rows_per_worker, 1, unroll=8, carry=idx0)(body)

    pltpu.sync_copy(st_v, out_hbm.at[pl.ds(row0, rows_per_worker), :])


def kernel(X):
    B, N, F = X.shape
    assert F == NFEAT
    rows = B * N
    rpw = rows // NUM_WORKERS
    assert rpw * NUM_WORKERS == rows and rpw % LANES == 0

    x_flat = X.reshape(rows * NFEAT)
    mesh = plsc.VectorSubcoreMesh(core_axis_name="c", subcore_axis_name="s")
    out2 = pl.kernel(
        functools.partial(_sc_body, rows_per_worker=rpw),
        out_type=jax.ShapeDtypeStruct((rows, NOUT), jnp.float32),
        mesh=mesh,
        compiler_params=pltpu.CompilerParams(
            needs_layout_passes=False, use_tc_tiling_on_sc=False
        ),
        scratch_types=[
            pltpu.VMEM((rpw * NFEAT + 2 * PAD,), jnp.float32),
            pltpu.VMEM((rpw, NOUT), jnp.float32),
        ],
    )(x_flat)
    return out2.reshape(B, N, NOUT)
